# Initial kernel scaffold; baseline (speedup 1.0000x reference)
import jax
import jax.numpy as jnp
from jax.experimental import pallas as pl


def kernel(user_idx, item_idx, user_emb, item_emb, user_bias, item_bias):
    raise NotImplementedError("write your pallas kernel here")


# speedup vs baseline: 1.1659x; 1.1659x over previous
import functools

import jax
import jax.numpy as jnp
from jax import lax
from jax.experimental import pallas as pl
from jax.experimental.pallas import tpu as pltpu
from jax.experimental.pallas import tpu_sc as plsc

BATCH = 16384
EMB = 128
L = 16
NC, NS = 2, 16
NW = NC * NS
R = BATCH // NW
C = 128
NCHUNK = R // C
GROUPS = C // L


_GATHER_DN = lax.GatherDimensionNumbers(
    offset_dims=(), collapsed_slice_dims=(0,), start_index_map=(0,))


def _shuffle(v, idx):
    return lax.gather(v, idx[:, None], _GATHER_DN, (1,),
                      mode=lax.GatherScatterMode.PROMISE_IN_BOUNDS)


def _body(uidx_hbm, iidx_hbm, uemb_hbm, iemb_hbm, ub_hbm, ib_hbm, out_hbm,
          uidx_v, iidx_v, urow_v, irow_v, ubv, ibv, out_v,
          sem_u, sem_i, sem_b, sem_c):
    cid = lax.axis_index("c")
    sid = lax.axis_index("s")
    wid = sid * NC + cid
    lanes = lax.iota(jnp.int32, L)

    pltpu.sync_copy(uidx_hbm.at[wid], uidx_v)
    pltpu.sync_copy(iidx_hbm.at[wid], iidx_v)

    zero = jnp.zeros((L,), jnp.float32)

    def chunk_body(ci, carry):
        cu = pltpu.async_copy(uemb_hbm.at[uidx_v.at[ci]], urow_v, sem_u)
        cv = pltpu.async_copy(iemb_hbm.at[iidx_v.at[ci]], irow_v, sem_i)
        cb = pltpu.async_copy(ub_hbm.at[uidx_v.at[ci]], ubv, sem_b)
        cc = pltpu.async_copy(ib_hbm.at[iidx_v.at[ci]], ibv, sem_c)
        cu.wait()
        cv.wait()
        cb.wait()
        cc.wait()

        def group_body(g, carry2):
            row0 = g * L
            acc = jnp.zeros((L,), jnp.float32)
            for r in range(L):
                row = row0 + r
                s = urow_v[row, pl.ds(0, L)] * irow_v[row, pl.ds(0, L)]
                for j in range(1, EMB // L):
                    s = s + (urow_v[row, pl.ds(j * L, L)] *
                             irow_v[row, pl.ds(j * L, L)])
                for sh in (8, 4, 2, 1):
                    s = s + _shuffle(s, lanes ^ sh)
                acc = jnp.where(lanes == r, s, acc)
            acc = acc + ubv[pl.ds(row0, L)] + ibv[pl.ds(row0, L)]
            out_v[pl.ds(ci * C + row0, L)] = jnp.maximum(acc, zero)
            return carry2

        lax.fori_loop(0, GROUPS, group_body, 0, unroll=False)
        return carry

    lax.fori_loop(0, NCHUNK, chunk_body, 0, unroll=False)

    pltpu.sync_copy(out_v, out_hbm.at[pl.ds(wid * R, R)])


@jax.jit
def kernel(user_idx, item_idx, user_emb, item_emb, user_bias, item_bias):
    uidx = user_idx.astype(jnp.int32).reshape(NW, NCHUNK, C)
    iidx = item_idx.astype(jnp.int32).reshape(NW, NCHUNK, C)
    ub = user_bias.reshape(-1)
    ib = item_bias.reshape(-1)

    mesh = plsc.VectorSubcoreMesh(core_axis_name="c", subcore_axis_name="s")
    run = pl.kernel(
        _body,
        out_type=jax.ShapeDtypeStruct((BATCH,), jnp.float32),
        mesh=mesh,
        scratch_types=[
            pltpu.VMEM((NCHUNK, C), jnp.int32),
            pltpu.VMEM((NCHUNK, C), jnp.int32),
            pltpu.VMEM((C, EMB), jnp.float32),
            pltpu.VMEM((C, EMB), jnp.float32),
            pltpu.VMEM((C,), jnp.float32),
            pltpu.VMEM((C,), jnp.float32),
            pltpu.VMEM((R,), jnp.float32),
            pltpu.SemaphoreType.DMA,
            pltpu.SemaphoreType.DMA,
            pltpu.SemaphoreType.DMA,
            pltpu.SemaphoreType.DMA,
        ],
    )
    return run(uidx, iidx, user_emb, item_emb, ub, ib)


# speedup vs baseline: 1.1905x; 1.0211x over previous
import jax
import jax.numpy as jnp
from jax import lax
from jax.experimental import pallas as pl
from jax.experimental.pallas import tpu as pltpu
from jax.experimental.pallas import tpu_sc as plsc

BATCH = 16384
EMB = 128
L = 16
NC, NS = 2, 16
NW = NC * NS
R = BATCH // NW
C = 128
NCHUNK = R // C
GROUPS = C // L

_GATHER_DN = lax.GatherDimensionNumbers(
    offset_dims=(), collapsed_slice_dims=(0,), start_index_map=(0,))


def _shuffle(v, idx):
    return lax.gather(v, idx[:, None], _GATHER_DN, (1,),
                      mode=lax.GatherScatterMode.PROMISE_IN_BOUNDS)


def _body(uidx_hbm, iidx_hbm, uemb_hbm, iemb_hbm, ub_hbm, ib_hbm, out_hbm,
          uidx_v, iidx_v, urow_v, irow_v, ubv, ibv, out_v, sems):
    cid = lax.axis_index("c")
    sid = lax.axis_index("s")
    wid = sid * NC + cid
    base = wid * R
    lanes = lax.iota(jnp.int32, L)
    zero = jnp.zeros((L,), jnp.float32)

    pltpu.sync_copy(uidx_hbm.at[pl.ds(base, R)], uidx_v)
    pltpu.sync_copy(iidx_hbm.at[pl.ds(base, R)], iidx_v)

    def start_chunk(ci, buf):
        uix = uidx_v.at[pl.ds(ci * C, C)]
        iix = iidx_v.at[pl.ds(ci * C, C)]
        pltpu.async_copy(uemb_hbm.at[uix], urow_v.at[buf], sems.at[buf, 0])
        pltpu.async_copy(iemb_hbm.at[iix], irow_v.at[buf], sems.at[buf, 1])
        pltpu.async_copy(ub_hbm.at[uix], ubv.at[buf], sems.at[buf, 2])
        pltpu.async_copy(ib_hbm.at[iix], ibv.at[buf], sems.at[buf, 3])

    def wait_chunk(buf):
        pltpu.make_async_copy(uemb_hbm.at[pl.ds(0, C)], urow_v.at[buf],
                              sems.at[buf, 0]).wait()
        pltpu.make_async_copy(iemb_hbm.at[pl.ds(0, C)], irow_v.at[buf],
                              sems.at[buf, 1]).wait()
        pltpu.make_async_copy(ub_hbm.at[pl.ds(0, C)], ubv.at[buf],
                              sems.at[buf, 2]).wait()
        pltpu.make_async_copy(ib_hbm.at[pl.ds(0, C)], ibv.at[buf],
                              sems.at[buf, 3]).wait()

    start_chunk(0, 0)
    for ci in range(NCHUNK):
        buf = ci % 2
        if ci + 1 < NCHUNK:
            start_chunk(ci + 1, 1 - buf)
        wait_chunk(buf)

        def group_body(g, carry2):
            row0 = g * L
            acc = jnp.zeros((L,), jnp.float32)
            for r in range(L):
                row = row0 + r
                s = (urow_v[buf, row, pl.ds(0, L)] *
                     irow_v[buf, row, pl.ds(0, L)])
                for j in range(1, EMB // L):
                    s = s + (urow_v[buf, row, pl.ds(j * L, L)] *
                             irow_v[buf, row, pl.ds(j * L, L)])
                for sh in (8, 4, 2, 1):
                    s = s + _shuffle(s, lanes ^ sh)
                acc = jnp.where(lanes == r, s, acc)
            acc = acc + ubv[buf, pl.ds(row0, L)] + ibv[buf, pl.ds(row0, L)]
            out_v[pl.ds(ci * C + row0, L)] = jnp.maximum(acc, zero)
            return carry2

        lax.fori_loop(0, GROUPS, group_body, 0, unroll=False)

    pltpu.sync_copy(out_v, out_hbm.at[pl.ds(base, R)])


@jax.jit
def kernel(user_idx, item_idx, user_emb, item_emb, user_bias, item_bias):
    mesh = plsc.VectorSubcoreMesh(core_axis_name="c", subcore_axis_name="s")
    run = pl.kernel(
        _body,
        out_type=jax.ShapeDtypeStruct((BATCH,), jnp.float32),
        mesh=mesh,
        scratch_types=[
            pltpu.VMEM((R,), jnp.int32),
            pltpu.VMEM((R,), jnp.int32),
            pltpu.VMEM((2, C, EMB), jnp.float32),
            pltpu.VMEM((2, C, EMB), jnp.float32),
            pltpu.VMEM((2, C), jnp.float32),
            pltpu.VMEM((2, C), jnp.float32),
            pltpu.VMEM((R,), jnp.float32),
            pltpu.SemaphoreType.DMA((2, 4)),
        ],
    )
    return run(user_idx.astype(jnp.int32), item_idx.astype(jnp.int32),
               user_emb, item_emb,
               user_bias.reshape(-1), item_bias.reshape(-1))
